# baseline (speedup 1.0000x reference)
import functools

import jax
import jax.numpy as jnp
from jax import lax
from jax.experimental import pallas as pl
from jax.experimental.pallas import tpu as pltpu, tpu_sc as plsc

BATCH = 16384
DIM = 64
ENT = 1_000_000
LANES = 128
SUB = 8
VROWS = ENT * DIM // LANES

_info = plsc.get_sparse_core_info()
_NC, _NS = _info.num_cores, _info.num_subcores
_NW = _NC * _NS
_BW = BATCH // _NW
_TCOL = _BW // LANES
_NCH = _BW // 16

_mesh = plsc.VectorSubcoreMesh(core_axis_name="c", subcore_axis_name="s")


@functools.partial(
    pl.kernel,
    mesh=_mesh,
    out_type=jax.ShapeDtypeStruct((DIM, BATCH), jnp.float32),
    scratch_types=[
        pltpu.VMEM((_BW,), jnp.int32),
        pltpu.VMEM((_BW,), jnp.int32),
        pltpu.VMEM((_BW, LANES), jnp.float32),
        pltpu.VMEM((DIM, _BW), jnp.float32),
        pltpu.SemaphoreType.DMA,
    ],
    compiler_params=pltpu.CompilerParams(
        use_tc_tiling_on_sc=True, needs_layout_passes=False
    ),
)
def _gather_kernel(idx_hbm, tbl_hbm, out_hbm, idx_v, hbuf, rows_v, outbuf, sem):
    w = lax.axis_index("s") * _NC + lax.axis_index("c")
    base = w * _BW
    pltpu.sync_copy(idx_hbm.at[pl.ds(base, _BW)], idx_v)

    @pl.loop(0, _NCH)
    def _half(k):
        r = idx_v[pl.ds(k * 16, 16)]
        hbuf[pl.ds(k * 16, 16)] = r >> 1

    pltpu.async_copy(tbl_hbm.at[hbuf], rows_v, sem).wait()

    @pl.loop(0, _TCOL)
    def _retile(tc):
        for lc in range(SUB):
            tgt = tc * LANES + lc * 16 + lax.iota(jnp.int32, 16)
            r = idx_v[pl.ds(tc * LANES + lc * 16, 16)]
            colbase = (r & 1) * DIM
            for c in range(DIM):
                val = plsc.load_gather(rows_v, [tgt, colbase + c])
                outbuf[c, pl.ds(tc * LANES + lc * 16, 16)] = val

    pltpu.sync_copy(outbuf, out_hbm.at[:, pl.ds(base, _BW)])


@jax.jit
def kernel(indexes, entity_table):
    tbl_v = jnp.reshape(entity_table, (VROWS, LANES))
    out_t = _gather_kernel(indexes.astype(jnp.int32), tbl_v)
    return out_t.T


# speedup vs baseline: 1.6814x; 1.6814x over previous
import functools

import jax
import jax.numpy as jnp
from jax import lax
from jax.experimental import pallas as pl
from jax.experimental.pallas import tpu as pltpu, tpu_sc as plsc

BATCH = 16384
DIM = 64
LANES = 128
SUB = 8

_info = plsc.get_sparse_core_info()
_NC, _NS = _info.num_cores, _info.num_subcores
_NW = _NC * _NS
_BW = BATCH // _NW
_TCOL = _BW // LANES

_CHUNK = 16
_N_CHUNKS = _BW // _CHUNK

_mesh = plsc.VectorSubcoreMesh(core_axis_name="c", subcore_axis_name="s")


@functools.partial(
    pl.kernel,
    mesh=_mesh,
    out_type=jax.ShapeDtypeStruct((DIM, BATCH), jnp.float32),
    scratch_types=[
        pltpu.VMEM((_BW,), jnp.int32),
        pltpu.VMEM((_BW, DIM), jnp.float32),
        pltpu.VMEM((DIM, _BW), jnp.float32),
        pltpu.SemaphoreType.DMA,
    ],
    compiler_params=pltpu.CompilerParams(
        use_tc_tiling_on_sc=True, needs_layout_passes=False
    ),
)
def _gather_kernel(idx_hbm, tbl_hbm, out_hbm, idx_v, rows_v, outbuf, sem):
    w = lax.axis_index("s") * _NC + lax.axis_index("c")
    base = w * _BW
    pltpu.sync_copy(idx_hbm.at[pl.ds(base, _BW)], idx_v)

    @pl.loop(0, _N_CHUNKS)
    def _issue(c):
        vec = idx_v[pl.ds(c * _CHUNK, _CHUNK)]
        for j in range(_CHUNK):
            r = vec[j]
            pltpu.async_copy(
                tbl_hbm.at[pl.ds(r, 1)],
                rows_v.at[pl.ds(c * _CHUNK + j, 1)],
                sem,
            )

    @pl.loop(0, _N_CHUNKS)
    def _drain(c):
        for j in range(_CHUNK):
            pltpu.make_async_copy(
                tbl_hbm.at[pl.ds(0, 1)],
                rows_v.at[pl.ds(c * _CHUNK + j, 1)],
                sem,
            ).wait()

    @pl.loop(0, _TCOL)
    def _retile(tc):
        for lc in range(SUB):
            tgt = tc * LANES + lc * 16 + lax.iota(jnp.int32, 16)
            for c in range(DIM):
                cvec = jnp.full((16,), c, jnp.int32)
                val = plsc.load_gather(rows_v, [tgt, cvec])
                outbuf[c, pl.ds(tc * LANES + lc * 16, 16)] = val

    pltpu.sync_copy(outbuf, out_hbm.at[:, pl.ds(base, _BW)])


@jax.jit
def kernel(indexes, entity_table):
    out_t = _gather_kernel(indexes.astype(jnp.int32), entity_table)
    return out_t.T


# speedup vs baseline: 1.7664x; 1.0506x over previous
import functools

import jax
import jax.numpy as jnp
from jax import lax
from jax.experimental import pallas as pl
from jax.experimental.pallas import tpu as pltpu, tpu_sc as plsc

BATCH = 16384
DIM = 64

_info = plsc.get_sparse_core_info()
_NC, _NS = _info.num_cores, _info.num_subcores
_NW = _NC * _NS
_B_PER_W = BATCH // _NW

_CHUNK = 16
_N_CHUNKS = _B_PER_W // _CHUNK

_mesh = plsc.VectorSubcoreMesh(core_axis_name="c", subcore_axis_name="s")


@functools.partial(
    pl.kernel,
    mesh=_mesh,
    out_type=jax.ShapeDtypeStruct((BATCH, DIM), jnp.float32),
    scratch_types=[
        pltpu.VMEM((_B_PER_W,), jnp.int32),
        pltpu.VMEM((_B_PER_W, DIM), jnp.float32),
        pltpu.SemaphoreType.DMA,
    ],
    compiler_params=pltpu.CompilerParams(use_tc_tiling_on_sc=True),
)
def _gather_kernel(idx_hbm, table_hbm, out_hbm, idx_v, rows_v, sem):
    wid = lax.axis_index("s") * _NC + lax.axis_index("c")
    base = wid * _B_PER_W
    pltpu.sync_copy(idx_hbm.at[pl.ds(base, _B_PER_W)], idx_v)

    @pl.loop(0, _N_CHUNKS)
    def _issue(c):
        vec = idx_v[pl.ds(c * _CHUNK, _CHUNK)]
        for j in range(_CHUNK):
            r = vec[j]
            pltpu.async_copy(
                table_hbm.at[pl.ds(r, 1)],
                rows_v.at[pl.ds(c * _CHUNK + j, 1)],
                sem,
            )

    @pl.loop(0, _N_CHUNKS)
    def _drain(c):
        for j in range(_CHUNK):
            pltpu.make_async_copy(
                table_hbm.at[pl.ds(0, 1)],
                rows_v.at[pl.ds(c * _CHUNK + j, 1)],
                sem,
            ).wait()

    pltpu.sync_copy(rows_v, out_hbm.at[pl.ds(base, _B_PER_W)])


@jax.jit
def kernel(indexes, entity_table):
    return _gather_kernel(indexes.astype(jnp.int32), entity_table)


# speedup vs baseline: 2.3906x; 1.3534x over previous
import functools

import jax
import jax.numpy as jnp
from jax import lax
from jax.experimental import pallas as pl
from jax.experimental.pallas import tpu as pltpu, tpu_sc as plsc

BATCH = 16384
DIM = 64
ENT = 1_000_000
LANES = 128

_info = plsc.get_sparse_core_info()
_NC, _NS = _info.num_cores, _info.num_subcores
_NW = _NC * _NS
_BW = BATCH // _NW
_NCH = _BW // 16
_SLOTS = 8

_mesh = plsc.VectorSubcoreMesh(core_axis_name="c", subcore_axis_name="s")


@functools.partial(
    pl.kernel,
    mesh=_mesh,
    out_type=jax.ShapeDtypeStruct((DIM, BATCH), jnp.float32),
    scratch_types=[
        pltpu.VMEM((_BW,), jnp.int32),
        pltpu.VMEM((_SLOTS, DIM, LANES), jnp.float32),
        pltpu.VMEM((DIM, _BW), jnp.float32),
        pltpu.SemaphoreType.DMA,
    ],
    compiler_params=pltpu.CompilerParams(
        use_tc_tiling_on_sc=True, needs_layout_passes=False
    ),
)
def _gather_kernel(idx_hbm, tbl_hbm, out_hbm, idx_v, buf, outbuf, sem):
    w = lax.axis_index("s") * _NC + lax.axis_index("c")
    base = pl.multiple_of(w * _BW, LANES)
    pltpu.sync_copy(idx_hbm.at[pl.ds(base, _BW)], idx_v)
    iota16 = lax.iota(jnp.int32, 16)

    @pl.loop(0, _NCH)
    def _chunk(cl):
        vec = idx_v[pl.ds(cl * 16, 16)]
        for half in range(2):
            for j in range(_SLOTS):
                r = vec[half * _SLOTS + j]
                lane0 = pl.multiple_of((r >> 7) * LANES, LANES)
                pltpu.async_copy(
                    tbl_hbm.at[:, pl.ds(lane0, LANES)], buf.at[j], sem
                )
            for j in range(_SLOTS):
                pltpu.make_async_copy(
                    tbl_hbm.at[:, pl.ds(0, LANES)], buf.at[j], sem
                ).wait()
            for j in range(_SLOTS):
                jj = half * _SLOTS + j
                r = vec[jj]
                lsplat = jnp.full((16,), r & 127, jnp.int32)
                jsplat = jnp.full((16,), j, jnp.int32)
                tsplat = jnp.full((16,), cl * 16 + jj, jnp.int32)
                for ck in range(DIM // 16):
                    cvec = ck * 16 + iota16
                    val = plsc.load_gather(buf, [jsplat, cvec, lsplat])
                    plsc.store_scatter(outbuf, [cvec, tsplat], val)

    pltpu.sync_copy(outbuf, out_hbm.at[:, pl.ds(base, _BW)])


@jax.jit
def kernel(indexes, entity_table):
    out_t = _gather_kernel(indexes.astype(jnp.int32), entity_table.T)
    return out_t.T


# speedup vs baseline: 2.4126x; 1.0092x over previous
import functools

import jax
import jax.numpy as jnp
from jax import lax
from jax.experimental import pallas as pl
from jax.experimental.pallas import tpu as pltpu, tpu_sc as plsc

BATCH = 16384
DIM = 64
ENT = 1_000_000
LANES = 128

_info = plsc.get_sparse_core_info()
_NC, _NS = _info.num_cores, _info.num_subcores
_NW = _NC * _NS
_BW = BATCH // _NW
_NCH = _BW // 16
_SLOTS = 8

_mesh = plsc.VectorSubcoreMesh(core_axis_name="c", subcore_axis_name="s")


@functools.partial(
    pl.kernel,
    mesh=_mesh,
    out_type=jax.ShapeDtypeStruct((DIM, BATCH), jnp.float32),
    scratch_types=[
        pltpu.VMEM((_BW,), jnp.int32),
        pltpu.VMEM((_SLOTS, DIM, LANES), jnp.float32),
        pltpu.VMEM((DIM, _BW), jnp.float32),
        pltpu.SemaphoreType.DMA,
        pltpu.SemaphoreType.DMA,
    ],
    compiler_params=pltpu.CompilerParams(
        use_tc_tiling_on_sc=True, needs_layout_passes=False
    ),
)
def _gather_kernel(idx_hbm, tbl_hbm, out_hbm, idx_v, buf, outbuf, semA, semB):
    w = lax.axis_index("s") * _NC + lax.axis_index("c")
    base = pl.multiple_of(w * _BW, LANES)
    pltpu.sync_copy(idx_hbm.at[pl.ds(base, _BW)], idx_v)
    iota16 = lax.iota(jnp.int32, 16)

    def _issue(vec, g, sem):
        sbase = (g & 1) * 4
        for t in range(4):
            r = vec[g * 4 + t]
            lane0 = pl.multiple_of((r >> 7) * LANES, LANES)
            pltpu.async_copy(
                tbl_hbm.at[:, pl.ds(lane0, LANES)], buf.at[sbase + t], sem
            )

    def _drain(g, sem):
        sbase = (g & 1) * 4
        for t in range(4):
            pltpu.make_async_copy(
                tbl_hbm.at[:, pl.ds(0, LANES)], buf.at[sbase + t], sem
            ).wait()

    def _extract(vec, cl, g):
        sbase = (g & 1) * 4
        for t in range(4):
            jj = g * 4 + t
            r = vec[jj]
            lsplat = jnp.full((16,), r & 127, jnp.int32)
            jsplat = jnp.full((16,), sbase + t, jnp.int32)
            tsplat = jnp.full((16,), cl * 16 + jj, jnp.int32)
            for ck in range(DIM // 16):
                cvec = ck * 16 + iota16
                val = plsc.load_gather(buf, [jsplat, cvec, lsplat])
                plsc.store_scatter(outbuf, [cvec, tsplat], val)

    @pl.loop(0, _NCH)
    def _chunk(cl):
        vec = idx_v[pl.ds(cl * 16, 16)]
        sems = (semA, semB)
        _issue(vec, 0, sems[0])
        for g in range(3):
            _issue(vec, g + 1, sems[(g + 1) & 1])
            _drain(g, sems[g & 1])
            _extract(vec, cl, g)
        _drain(3, sems[1])
        _extract(vec, cl, 3)

    pltpu.sync_copy(outbuf, out_hbm.at[:, pl.ds(base, _BW)])


@jax.jit
def kernel(indexes, entity_table):
    out_t = _gather_kernel(indexes.astype(jnp.int32), entity_table.T)
    return out_t.T
